# Initial kernel scaffold; baseline (speedup 1.0000x reference)
import jax
import jax.numpy as jnp
from jax.experimental import pallas as pl


def kernel(hidden_states, seg_indexs, W, b):
    raise NotImplementedError("write your pallas kernel here")


# speedup vs baseline: 1.2570x; 1.2570x over previous
import functools

import jax
import jax.numpy as jnp
from jax import lax
from jax.experimental import pallas as pl
from jax.experimental.pallas import tpu as pltpu
from jax.experimental.pallas import tpu_sc as plsc

B, S, H, L = 16, 4096, 1024, 2048
NW = 32
IDX_PER_W = L * B // NW
CH = 32
G = IDX_PER_W // CH
LANES = 16
HV = H // LANES


def _pool_body(seg_hbm, hidden_hbm, out_hbm, idx_v, buf_a, buf_b, row_v,
               sem_a, sem_b):
    wid = lax.axis_index("s") * 2 + lax.axis_index("c")
    base = (wid // 2) * S

    pltpu.sync_copy(seg_hbm.at[wid], idx_v)
    for g in range(G):
        for c in range(CH // LANES):
            sl = pl.ds(c * LANES, LANES)
            idx_v[g, sl] = idx_v[g, sl] + base

    for j in range(HV):
        row_v[pl.ds(j * LANES, LANES)] = jnp.zeros((LANES,), jnp.float32)

    bufs = (buf_a, buf_b)
    sems = (sem_a, sem_b)

    def _acc_from(buf):
        def body(j, carry):
            sl = pl.ds(j * LANES, LANES)
            v = buf[0, sl]
            for r in range(1, CH):
                v = v + buf[r, sl]
            row_v[sl] = row_v[sl] + v
            return carry

        lax.fori_loop(0, HV, body, 0)

    pending = [
        pltpu.async_copy(hidden_hbm.at[idx_v.at[0]], buf_a, sem_a),
        pltpu.async_copy(hidden_hbm.at[idx_v.at[1]], buf_b, sem_b),
    ]
    for g in range(G):
        p = g % 2
        pending[p].wait()
        _acc_from(bufs[p])
        if g + 2 < G:
            pending[p] = pltpu.async_copy(
                hidden_hbm.at[idx_v.at[g + 2]], bufs[p], sems[p])

    pltpu.sync_copy(row_v, out_hbm.at[wid])


_pool = functools.partial(
    pl.kernel,
    out_type=jax.ShapeDtypeStruct((NW, H), jnp.float32),
    mesh=plsc.VectorSubcoreMesh(core_axis_name="c", subcore_axis_name="s"),
    scratch_types=[
        pltpu.VMEM((G, CH), jnp.int32),
        pltpu.VMEM((CH, H), jnp.float32),
        pltpu.VMEM((CH, H), jnp.float32),
        pltpu.VMEM((H,), jnp.float32),
        pltpu.SemaphoreType.DMA,
        pltpu.SemaphoreType.DMA,
    ],
)(_pool_body)


def _dense_body(p_ref, w_ref, b_ref, o_ref):
    m = (p_ref[:, 0, :] + p_ref[:, 1, :]) * (1.0 / L)
    y = lax.dot_general(m, w_ref[...], (((1,), (1,)), ((), ())),
                        preferred_element_type=jnp.float32)
    o_ref[...] = jnp.tanh(y + b_ref[...])


def kernel(hidden_states, seg_indexs, W, b):
    seg = seg_indexs.astype(jnp.int32).reshape(NW, G, CH)
    hidden_flat = hidden_states.reshape(B * S, H)
    partials = _pool(seg, hidden_flat)
    return pl.pallas_call(
        _dense_body,
        out_shape=jax.ShapeDtypeStruct((B, H), jnp.float32),
    )(partials.reshape(B, 2, H), W, b.reshape(1, H))


# speedup vs baseline: 1.6723x; 1.3304x over previous
import functools

import jax
import jax.numpy as jnp
from jax import lax
from jax.experimental import pallas as pl
from jax.experimental.pallas import tpu as pltpu
from jax.experimental.pallas import tpu_sc as plsc

B, S, H, L = 16, 4096, 1024, 2048
NW = 32
IDX_PER_W = L * B // NW
CH = 32
G = IDX_PER_W // CH
LANES = 16
HV = H // LANES


def _pool_body(seg_hbm, hidden_hbm, out_hbm, idx_v, buf_a, buf_b, row_v,
               sem_a, sem_b):
    wid = lax.axis_index("s") * 2 + lax.axis_index("c")
    base = (wid // 2) * S

    pltpu.sync_copy(seg_hbm.at[wid], idx_v)
    for g in range(G):
        for c in range(CH // LANES):
            sl = pl.ds(c * LANES, LANES)
            idx_v[g, sl] = idx_v[g, sl] + base

    for j in range(HV):
        row_v[pl.ds(j * LANES, LANES)] = jnp.zeros((LANES,), jnp.float32)

    bufs = (buf_a, buf_b)
    sems = (sem_a, sem_b)

    def _acc_from(buf):
        @plsc.parallel_loop(0, HV)
        def _(j):
            sl = pl.ds(j * LANES, LANES)
            v = buf[0, sl]
            for r in range(1, CH):
                v = v + buf[r, sl]
            row_v[sl] = row_v[sl] + v

    pending = [
        pltpu.async_copy(hidden_hbm.at[idx_v.at[0]], buf_a, sem_a),
        pltpu.async_copy(hidden_hbm.at[idx_v.at[1]], buf_b, sem_b),
    ]
    for g in range(G):
        p = g % 2
        pending[p].wait()
        _acc_from(bufs[p])
        if g + 2 < G:
            pending[p] = pltpu.async_copy(
                hidden_hbm.at[idx_v.at[g + 2]], bufs[p], sems[p])

    pltpu.sync_copy(row_v, out_hbm.at[wid])


_pool = functools.partial(
    pl.kernel,
    out_type=jax.ShapeDtypeStruct((NW, H), jnp.float32),
    mesh=plsc.VectorSubcoreMesh(core_axis_name="c", subcore_axis_name="s"),
    scratch_types=[
        pltpu.VMEM((G, CH), jnp.int32),
        pltpu.VMEM((CH, H), jnp.float32),
        pltpu.VMEM((CH, H), jnp.float32),
        pltpu.VMEM((H,), jnp.float32),
        pltpu.SemaphoreType.DMA,
        pltpu.SemaphoreType.DMA,
    ],
)(_pool_body)


def _dense_body(p_ref, w_ref, b_ref, o_ref):
    m = (p_ref[:, 0, :] + p_ref[:, 1, :]) * (1.0 / L)
    y = lax.dot_general(m, w_ref[...], (((1,), (1,)), ((), ())),
                        preferred_element_type=jnp.float32)
    o_ref[...] = jnp.tanh(y + b_ref[...])


def kernel(hidden_states, seg_indexs, W, b):
    seg = seg_indexs.astype(jnp.int32).reshape(NW, G, CH)
    hidden_flat = hidden_states.reshape(B * S, H)
    partials = _pool(seg, hidden_flat)
    return pl.pallas_call(
        _dense_body,
        out_shape=jax.ShapeDtypeStruct((B, H), jnp.float32),
    )(partials.reshape(B, 2, H), W, b.reshape(1, H))
